# Initial kernel scaffold; baseline (speedup 1.0000x reference)
import jax
import jax.numpy as jnp
from jax.experimental import pallas as pl


def kernel(uv, n, v, emb0, emb1, emb2, emb3, lm_W0, lm_b0, lm_W1, lm_b1, lm_W2, lm_b2, lm_W3, lm_b3, rf_W0, rf_b0, rf_W1, rf_b1, rf_W2, rf_b2, rf_W3, rf_b3):
    raise NotImplementedError("write your pallas kernel here")


# speedup vs baseline: 37.6512x; 37.6512x over previous
import functools

import jax
import jax.numpy as jnp
import numpy as np
from jax import lax
from jax.experimental import pallas as pl
from jax.experimental.pallas import tpu as pltpu
from jax.experimental.pallas import tpu_sc as plsc

C0 = 0.28209479177387814
C1 = 0.4886025119029199
C2 = [1.0925484305920792, -1.0925484305920792, 0.31539156525252005,
      -1.0925484305920792, 0.5462742152960396]
C3 = [-0.5900435899266435, 2.890611442640554, -0.4570457994644658,
      0.3731763325901154, -0.4570457994644658, 1.445305721320277,
      -0.5900435899266435]

_PRIME = int(np.uint32(2654435761).view(np.int32))
_MASK = (1 << 16) - 1
_HI16 = int(np.uint32(0xFFFF0000).view(np.int32))

_NLEVELS = 4
_TABLE = 1 << 16
_NW = 32
_BLK = 4096


def _sc_hashgrid(u_flat, v_flat, tabs, n_pix):
  chunk = n_pix // (_NW // _NLEVELS)

  mesh = plsc.VectorSubcoreMesh(core_axis_name="c", subcore_axis_name="s")

  @functools.partial(
      pl.kernel,
      out_type=jax.ShapeDtypeStruct((2 * _NLEVELS, n_pix), jnp.float32),
      mesh=mesh,
      compiler_params=pltpu.CompilerParams(needs_layout_passes=False),
      scratch_types=[
          pltpu.VMEM((_TABLE,), jnp.int32),
          pltpu.VMEM((_BLK,), jnp.float32),
          pltpu.VMEM((_BLK,), jnp.float32),
          pltpu.VMEM((_BLK,), jnp.float32),
          pltpu.VMEM((_BLK,), jnp.float32),
      ],
  )
  def sc_kernel(u_hbm, v_hbm, tabs_hbm, out_hbm, tab_v, u_v, v_v, e0_v, e1_v):
    cid = lax.axis_index("c")
    sid = lax.axis_index("s")
    wid = sid * 2 + cid
    level = wid % _NLEVELS
    cidx = wid // _NLEVELS
    base = cidx * chunk
    res = (jnp.int32(1) << (level + 9)).astype(jnp.float32)

    pltpu.sync_copy(tabs_hbm.at[level], tab_v)

    def blk_body(b, carry):
      gbase = base + b * _BLK
      pltpu.sync_copy(u_hbm.at[pl.ds(gbase, _BLK)], u_v)
      pltpu.sync_copy(v_hbm.at[pl.ds(gbase, _BLK)], v_v)

      def vec_body(i, c2):
        off = i * 16
        su = u_v[pl.ds(off, 16)] * res
        sv = v_v[pl.ds(off, 16)] * res
        iu = su.astype(jnp.int32)
        iv = sv.astype(jnp.int32)
        fu = su - iu.astype(jnp.float32)
        fv = sv - iv.astype(jnp.float32)
        hv0 = iv * _PRIME
        hv1 = hv0 + _PRIME
        iu1 = iu + 1
        h00 = (iu ^ hv0) & _MASK
        h01 = (iu ^ hv1) & _MASK
        h10 = (iu1 ^ hv0) & _MASK
        h11 = (iu1 ^ hv1) & _MASK

        def rows(h):
          r = plsc.load_gather(tab_v, [h])
          a = plsc.bitcast(r << 16, jnp.float32)
          b_ = plsc.bitcast(r & _HI16, jnp.float32)
          return a, b_

        a00, b00 = rows(h00)
        a01, b01 = rows(h01)
        a10, b10 = rows(h10)
        a11, b11 = rows(h11)
        gu = 1.0 - fu
        gv = 1.0 - fv
        w00 = gu * gv
        w01 = gu * fv
        w10 = fu * gv
        w11 = fu * fv
        e0_v[pl.ds(off, 16)] = a00 * w00 + a01 * w01 + a10 * w10 + a11 * w11
        e1_v[pl.ds(off, 16)] = b00 * w00 + b01 * w01 + b10 * w10 + b11 * w11
        return c2

      lax.fori_loop(0, _BLK // 16, vec_body, 0, unroll=4)
      pltpu.sync_copy(e0_v, out_hbm.at[2 * level, pl.ds(gbase, _BLK)])
      pltpu.sync_copy(e1_v, out_hbm.at[2 * level + 1, pl.ds(gbase, _BLK)])
      return carry

    lax.fori_loop(0, chunk // _BLK, blk_body, 0)

  return sc_kernel(u_flat, v_flat, tabs)


def _sh16(x, y, z):
  xx, yy, zz = x * x, y * y, z * z
  xy, yz, xz = x * y, y * z, x * z
  comps = [
      jnp.full_like(x, C0), -C1 * y, C1 * z, -C1 * x,
      C2[0] * xy, C2[1] * yz, C2[2] * (2.0 * zz - xx - yy),
      C2[3] * xz, C2[4] * (xx - yy),
      C3[0] * y * (3.0 * xx - yy), C3[1] * xy * z,
      C3[2] * y * (4.0 * zz - xx - yy),
      C3[3] * z * (2.0 * zz - 3.0 * xx - 3.0 * yy),
      C3[4] * x * (4.0 * zz - xx - yy),
      C3[5] * z * (xx - yy), C3[6] * x * (xx - 3.0 * yy),
  ]
  return jnp.concatenate(comps, axis=0)


_PB = 2048

_SIN_C = (3.141591396703514, -5.167677423202123, 2.549879336105666,
          -0.5982788113360435, 0.08047606178445821, -0.005990654268057893)


def _sinpi(z):
  k2 = jnp.floor(0.5 * z + 0.5)
  r = z - 2.0 * k2
  s = r * r
  p = _SIN_C[5]
  for c in _SIN_C[4::-1]:
    p = p * s + c
  return r * p


def _dot(w, x):
  return jax.lax.dot_general(
      w, x, (((1,), (0,)), ((), ())),
      precision=jax.lax.Precision.HIGHEST,
      preferred_element_type=jnp.float32)


def _tc_mlp(nv8, feats, lm_Ws, lm_bs, rf_Ws, rf_bs, n_pix):
  w0s = [20.0, 1.0, 1.0, 1.0]

  def body(nv_ref, ft_ref,
           w0, b0, w1, b1, w2, b2, w3, b3,
           r0, s0, r1, s1, r2, s2, r3, s3,
           out_ref):
    nv = nv_ref[...]
    n_sh = _sh16(nv[0:1], nv[1:2], nv[2:3])
    v_sh = _sh16(nv[3:4], nv[4:5], nv[5:6])
    h = jnp.concatenate([n_sh, ft_ref[...]], axis=0)
    for wr, br in zip((w0, w1, w2, w3), (b0, b1, b2, b3)):
      h = _sinpi(_dot(wr[...], h) + br[...])
    h = jnp.concatenate([h, v_sh], axis=0)
    for wr, br in zip((r0, r1, r2), (s0, s1, s2)):
      h = _sinpi(_dot(wr[...], h) + br[...])
    z = _dot(r3[...], h) + s3[...]
    out_ref[...] = 1.0 / (1.0 + jnp.exp(-z))

  grid = (n_pix // _PB,)
  full = lambda a: pl.BlockSpec(a.shape, lambda i: (0,) * a.ndim)
  wspecs = []
  wargs = []
  inv_pi = 1.0 / np.pi
  scales = [w * inv_pi for w in w0s] + [inv_pi, inv_pi, inv_pi, 1.0]
  for (W, b), sc in zip(list(zip(lm_Ws, lm_bs)) + list(zip(rf_Ws, rf_bs)),
                        scales):
    b2 = (b * sc).reshape(-1, 1).astype(jnp.float32)
    wargs += [(W * sc).astype(jnp.float32), b2]
    wspecs += [full(W), full(b2)]

  return pl.pallas_call(
      body,
      grid=grid,
      in_specs=[
          pl.BlockSpec((8, _PB), lambda i: (0, i)),
          pl.BlockSpec((8, _PB), lambda i: (0, i)),
          *wspecs,
      ],
      out_specs=pl.BlockSpec((3, _PB), lambda i: (0, i)),
      out_shape=jax.ShapeDtypeStruct((3, n_pix), jnp.float32),
  )(nv8, feats, *wargs)


def _pack_table(emb):
  b16 = lax.bitcast_convert_type(emb.astype(jnp.bfloat16), jnp.uint16)
  word = b16[:, 0].astype(jnp.uint32) | (b16[:, 1].astype(jnp.uint32) << 16)
  return lax.bitcast_convert_type(word, jnp.int32)


def kernel(uv, n, v, emb0, emb1, emb2, emb3,
           lm_W0, lm_b0, lm_W1, lm_b1, lm_W2, lm_b2, lm_W3, lm_b3,
           rf_W0, rf_b0, rf_W1, rf_b1, rf_W2, rf_b2, rf_W3, rf_b3):
  B, W, H = uv.shape[:3]
  n_pix = B * W * H

  u_flat = uv[..., 0].reshape(n_pix)
  v_flat = uv[..., 1].reshape(n_pix)
  tabs = jnp.stack([_pack_table(e) for e in (emb0, emb1, emb2, emb3)])

  feats = _sc_hashgrid(u_flat, v_flat, tabs, n_pix)

  nT = n.reshape(n_pix, 3).T
  vT = v.reshape(n_pix, 3).T
  nv8 = jnp.concatenate([nT, vT, jnp.zeros((2, n_pix), jnp.float32)], axis=0)

  out = _tc_mlp(nv8, feats,
                (lm_W0, lm_W1, lm_W2, lm_W3), (lm_b0, lm_b1, lm_b2, lm_b3),
                (rf_W0, rf_W1, rf_W2, rf_W3), (rf_b0, rf_b1, rf_b2, rf_b3),
                n_pix)
  return out.T.reshape(B, W, H, 3)


# speedup vs baseline: 46.0613x; 1.2234x over previous
import functools

import jax
import jax.numpy as jnp
import numpy as np
from jax import lax
from jax.experimental import pallas as pl
from jax.experimental.pallas import tpu as pltpu
from jax.experimental.pallas import tpu_sc as plsc

C0 = 0.28209479177387814
C1 = 0.4886025119029199
C2 = [1.0925484305920792, -1.0925484305920792, 0.31539156525252005,
      -1.0925484305920792, 0.5462742152960396]
C3 = [-0.5900435899266435, 2.890611442640554, -0.4570457994644658,
      0.3731763325901154, -0.4570457994644658, 1.445305721320277,
      -0.5900435899266435]

_PRIME = int(np.uint32(2654435761).view(np.int32))
_MASK = (1 << 16) - 1
_HI16 = int(np.uint32(0xFFFF0000).view(np.int32))

_NLEVELS = 4
_TABLE = 1 << 16
_NW = 32
_BLK = 4096


def _sc_hashgrid(u_flat, v_flat, tabs, n_pix):
  chunk = n_pix // (_NW // _NLEVELS)

  mesh = plsc.VectorSubcoreMesh(core_axis_name="c", subcore_axis_name="s")

  @functools.partial(
      pl.kernel,
      out_type=jax.ShapeDtypeStruct((2 * _NLEVELS, n_pix), jnp.float32),
      mesh=mesh,
      compiler_params=pltpu.CompilerParams(needs_layout_passes=False),
      scratch_types=[
          pltpu.VMEM((_TABLE,), jnp.int32),
          pltpu.VMEM((_BLK,), jnp.float32),
          pltpu.VMEM((_BLK,), jnp.float32),
          pltpu.VMEM((_BLK,), jnp.float32),
          pltpu.VMEM((_BLK,), jnp.float32),
      ],
  )
  def sc_kernel(u_hbm, v_hbm, tabs_hbm, out_hbm, tab_v, u_v, v_v, e0_v, e1_v):
    cid = lax.axis_index("c")
    sid = lax.axis_index("s")
    wid = sid * 2 + cid
    level = wid % _NLEVELS
    cidx = wid // _NLEVELS
    base = cidx * chunk
    res = (jnp.int32(1) << (level + 9)).astype(jnp.float32)

    pltpu.sync_copy(tabs_hbm.at[level], tab_v)

    def blk_body(b, carry):
      gbase = base + b * _BLK
      pltpu.sync_copy(u_hbm.at[pl.ds(gbase, _BLK)], u_v)
      pltpu.sync_copy(v_hbm.at[pl.ds(gbase, _BLK)], v_v)

      @plsc.parallel_loop(0, _BLK // 16, unroll=8)
      def vec_body(i):
        off = i * 16
        su = u_v[pl.ds(off, 16)] * res
        sv = v_v[pl.ds(off, 16)] * res
        iu = su.astype(jnp.int32)
        iv = sv.astype(jnp.int32)
        fu = su - iu.astype(jnp.float32)
        fv = sv - iv.astype(jnp.float32)
        hv0 = iv * _PRIME
        hv1 = hv0 + _PRIME
        iu1 = iu + 1
        h00 = (iu ^ hv0) & _MASK
        h01 = (iu ^ hv1) & _MASK
        h10 = (iu1 ^ hv0) & _MASK
        h11 = (iu1 ^ hv1) & _MASK

        def rows(h):
          r = plsc.load_gather(tab_v, [h])
          a = plsc.bitcast(r << 16, jnp.float32)
          b_ = plsc.bitcast(r & _HI16, jnp.float32)
          return a, b_

        a00, b00 = rows(h00)
        a01, b01 = rows(h01)
        a10, b10 = rows(h10)
        a11, b11 = rows(h11)
        gu = 1.0 - fu
        gv = 1.0 - fv
        w00 = gu * gv
        w01 = gu * fv
        w10 = fu * gv
        w11 = fu * fv
        e0_v[pl.ds(off, 16)] = a00 * w00 + a01 * w01 + a10 * w10 + a11 * w11
        e1_v[pl.ds(off, 16)] = b00 * w00 + b01 * w01 + b10 * w10 + b11 * w11
      pltpu.sync_copy(e0_v, out_hbm.at[2 * level, pl.ds(gbase, _BLK)])
      pltpu.sync_copy(e1_v, out_hbm.at[2 * level + 1, pl.ds(gbase, _BLK)])
      return carry

    lax.fori_loop(0, chunk // _BLK, blk_body, 0)

  return sc_kernel(u_flat, v_flat, tabs)


def _sh16(x, y, z):
  xx, yy, zz = x * x, y * y, z * z
  xy, yz, xz = x * y, y * z, x * z
  comps = [
      jnp.full_like(x, C0), -C1 * y, C1 * z, -C1 * x,
      C2[0] * xy, C2[1] * yz, C2[2] * (2.0 * zz - xx - yy),
      C2[3] * xz, C2[4] * (xx - yy),
      C3[0] * y * (3.0 * xx - yy), C3[1] * xy * z,
      C3[2] * y * (4.0 * zz - xx - yy),
      C3[3] * z * (2.0 * zz - 3.0 * xx - 3.0 * yy),
      C3[4] * x * (4.0 * zz - xx - yy),
      C3[5] * z * (xx - yy), C3[6] * x * (xx - 3.0 * yy),
  ]
  return jnp.concatenate(comps, axis=0)


_PB = 4096

_SIN_C = (3.141591396703514, -5.167677423202123, 2.549879336105666,
          -0.5982788113360435, 0.08047606178445821, -0.005990654268057893)


def _sinpi(z):
  k2 = jnp.floor(0.5 * z + 0.5)
  r = z - 2.0 * k2
  s = r * r
  p = _SIN_C[5]
  for c in _SIN_C[4::-1]:
    p = p * s + c
  return r * p


def _dot(w, x, precision=jax.lax.Precision.DEFAULT):
  return jax.lax.dot_general(
      w, x, (((1,), (0,)), ((), ())),
      precision=precision,
      preferred_element_type=jnp.float32)


def _tc_mlp(z0T, v3, lm_Ws, lm_bs, rf_Ws, rf_bs, n_pix):
  def body(z0_ref, v_ref,
           w1, b1, w2, b2, w3, b3,
           r0, s0, r1, s1, r2, s2, r3, s3,
           out_ref):
    vv = v_ref[...]
    v_sh = _sh16(vv[0:1], vv[1:2], vv[2:3])
    h = _sinpi(z0_ref[...] * (20.0 / np.pi))
    for wr, br in zip((w1, w2, w3), (b1, b2, b3)):
      h = _sinpi(_dot(wr[...], h) + br[...])
    h = jnp.concatenate([h, v_sh], axis=0)
    for wr, br in zip((r0, r1, r2), (s0, s1, s2)):
      h = _sinpi(_dot(wr[...], h) + br[...])
    z = _dot(r3[...], h) + s3[...]
    out_ref[...] = 1.0 / (1.0 + jnp.exp(-z))

  grid = (n_pix // _PB,)
  full = lambda a: pl.BlockSpec(a.shape, lambda i: (0,) * a.ndim)
  wargs = []
  wspecs = []
  inv_pi = 1.0 / np.pi
  scales = [inv_pi] * 6 + [1.0]
  for (W, b), sc in zip(list(zip(lm_Ws[1:], lm_bs[1:]))
                        + list(zip(rf_Ws, rf_bs)), scales):
    b2 = (b * sc).reshape(-1, 1).astype(jnp.float32)
    wargs += [(W * sc).astype(jnp.float32), b2]
    wspecs += [full(W), full(b2)]

  return pl.pallas_call(
      body,
      grid=grid,
      in_specs=[
          pl.BlockSpec((64, _PB), lambda i: (0, i)),
          pl.BlockSpec((3, _PB), lambda i: (0, i)),
          *wspecs,
      ],
      out_specs=pl.BlockSpec((3, _PB), lambda i: (0, i)),
      out_shape=jax.ShapeDtypeStruct((3, n_pix), jnp.float32),
  )(z0T, v3, *wargs)


def _pack_table(emb):
  b16 = lax.bitcast_convert_type(emb.astype(jnp.bfloat16), jnp.uint16)
  word = b16[:, 0].astype(jnp.uint32) | (b16[:, 1].astype(jnp.uint32) << 16)
  return lax.bitcast_convert_type(word, jnp.int32)


def kernel(uv, n, v, emb0, emb1, emb2, emb3,
           lm_W0, lm_b0, lm_W1, lm_b1, lm_W2, lm_b2, lm_W3, lm_b3,
           rf_W0, rf_b0, rf_W1, rf_b1, rf_W2, rf_b2, rf_W3, rf_b3):
  B, W, H = uv.shape[:3]
  n_pix = B * W * H

  u_flat = uv[..., 0].reshape(n_pix)
  v_flat = uv[..., 1].reshape(n_pix)
  tabs = jnp.stack([_pack_table(e) for e in (emb0, emb1, emb2, emb3)])

  feats = _sc_hashgrid(u_flat, v_flat, tabs, n_pix)

  nT = n.reshape(n_pix, 3).T
  vT = v.reshape(n_pix, 3).T
  n_shT = _sh16(nT[0:1], nT[1:2], nT[2:3])
  x0T = jnp.concatenate([n_shT, feats], axis=0)
  z0T = lm_W0 @ x0T + lm_b0[:, None]

  out = _tc_mlp(z0T, vT,
                (lm_W0, lm_W1, lm_W2, lm_W3), (lm_b0, lm_b1, lm_b2, lm_b3),
                (rf_W0, rf_W1, rf_W2, rf_W3), (rf_b0, rf_b1, rf_b2, rf_b3),
                n_pix)
  return out.T.reshape(B, W, H, 3)


# speedup vs baseline: 47.9613x; 1.0412x over previous
import functools

import jax
import jax.numpy as jnp
import numpy as np
from jax import lax
from jax.experimental import pallas as pl
from jax.experimental.pallas import tpu as pltpu
from jax.experimental.pallas import tpu_sc as plsc

C0 = 0.28209479177387814
C1 = 0.4886025119029199
C2 = [1.0925484305920792, -1.0925484305920792, 0.31539156525252005,
      -1.0925484305920792, 0.5462742152960396]
C3 = [-0.5900435899266435, 2.890611442640554, -0.4570457994644658,
      0.3731763325901154, -0.4570457994644658, 1.445305721320277,
      -0.5900435899266435]

_PRIME = int(np.uint32(2654435761).view(np.int32))
_MASK = (1 << 16) - 1
_HI16 = int(np.uint32(0xFFFF0000).view(np.int32))

_NLEVELS = 4
_TABLE = 1 << 16
_NW = 32
_BLK = 4096


def _sc_hashgrid(u_flat, v_flat, tabs, n_pix):
  chunk = n_pix // (_NW // _NLEVELS)

  mesh = plsc.VectorSubcoreMesh(core_axis_name="c", subcore_axis_name="s")

  @functools.partial(
      pl.kernel,
      out_type=jax.ShapeDtypeStruct((2 * _NLEVELS, n_pix), jnp.float32),
      mesh=mesh,
      compiler_params=pltpu.CompilerParams(needs_layout_passes=False),
      scratch_types=[
          pltpu.VMEM((_TABLE,), jnp.int32),
          pltpu.VMEM((_BLK,), jnp.float32),
          pltpu.VMEM((_BLK,), jnp.float32),
          pltpu.VMEM((_BLK,), jnp.float32),
          pltpu.VMEM((_BLK,), jnp.float32),
      ],
  )
  def sc_kernel(u_hbm, v_hbm, tabs_hbm, out_hbm, tab_v, u_v, v_v, e0_v, e1_v):
    cid = lax.axis_index("c")
    sid = lax.axis_index("s")
    wid = sid * 2 + cid
    level = wid % _NLEVELS
    cidx = wid // _NLEVELS
    base = cidx * chunk
    res = (jnp.int32(1) << (level + 9)).astype(jnp.float32)

    pltpu.sync_copy(tabs_hbm.at[level], tab_v)

    def blk_body(b, carry):
      gbase = base + b * _BLK
      pltpu.sync_copy(u_hbm.at[pl.ds(gbase, _BLK)], u_v)
      pltpu.sync_copy(v_hbm.at[pl.ds(gbase, _BLK)], v_v)

      @plsc.parallel_loop(0, _BLK // 16, unroll=8)
      def vec_body(i):
        off = i * 16
        su = u_v[pl.ds(off, 16)] * res
        sv = v_v[pl.ds(off, 16)] * res
        iu = su.astype(jnp.int32)
        iv = sv.astype(jnp.int32)
        fu = su - iu.astype(jnp.float32)
        fv = sv - iv.astype(jnp.float32)
        hv0 = iv * _PRIME
        hv1 = hv0 + _PRIME
        iu1 = iu + 1
        h00 = (iu ^ hv0) & _MASK
        h01 = (iu ^ hv1) & _MASK
        h10 = (iu1 ^ hv0) & _MASK
        h11 = (iu1 ^ hv1) & _MASK

        def rows(h):
          r = plsc.load_gather(tab_v, [h])
          a = plsc.bitcast(r << 16, jnp.float32)
          b_ = plsc.bitcast(r & _HI16, jnp.float32)
          return a, b_

        a00, b00 = rows(h00)
        a01, b01 = rows(h01)
        a10, b10 = rows(h10)
        a11, b11 = rows(h11)
        gu = 1.0 - fu
        gv = 1.0 - fv
        w00 = gu * gv
        w01 = gu * fv
        w10 = fu * gv
        w11 = fu * fv
        e0_v[pl.ds(off, 16)] = a00 * w00 + a01 * w01 + a10 * w10 + a11 * w11
        e1_v[pl.ds(off, 16)] = b00 * w00 + b01 * w01 + b10 * w10 + b11 * w11
      pltpu.sync_copy(e0_v, out_hbm.at[2 * level, pl.ds(gbase, _BLK)])
      pltpu.sync_copy(e1_v, out_hbm.at[2 * level + 1, pl.ds(gbase, _BLK)])
      return carry

    lax.fori_loop(0, chunk // _BLK, blk_body, 0)

  return sc_kernel(u_flat, v_flat, tabs)


def _sh16(x, y, z):
  xx, yy, zz = x * x, y * y, z * z
  xy, yz, xz = x * y, y * z, x * z
  comps = [
      jnp.full_like(x, C0), -C1 * y, C1 * z, -C1 * x,
      C2[0] * xy, C2[1] * yz, C2[2] * (2.0 * zz - xx - yy),
      C2[3] * xz, C2[4] * (xx - yy),
      C3[0] * y * (3.0 * xx - yy), C3[1] * xy * z,
      C3[2] * y * (4.0 * zz - xx - yy),
      C3[3] * z * (2.0 * zz - 3.0 * xx - 3.0 * yy),
      C3[4] * x * (4.0 * zz - xx - yy),
      C3[5] * z * (xx - yy), C3[6] * x * (xx - 3.0 * yy),
  ]
  return jnp.concatenate(comps, axis=0)


_PB = 4096

_SIN_C = (3.1415270439721206, -5.166390368574228, 2.54267183018941,
          -0.5818045120988824, 0.06400176254729995)


def _sinpi(z):
  k2 = jnp.floor(0.5 * z + 0.5)
  r = z - 2.0 * k2
  s = r * r
  p = _SIN_C[-1]
  for c in _SIN_C[-2::-1]:
    p = p * s + c
  return r * p


def _dot(w, x, precision=jax.lax.Precision.DEFAULT):
  return jax.lax.dot_general(
      w, x, (((1,), (0,)), ((), ())),
      precision=precision,
      preferred_element_type=jnp.float32)


def _tc_mlp(z0T, v3, lm_Ws, lm_bs, rf_Ws, rf_bs, n_pix):
  def body(z0_ref, v_ref,
           w1, b1, w2, b2, w3, b3,
           r0, s0, r1, s1, r2, s2, r3, s3,
           out_ref):
    vv = v_ref[...]
    v_sh = _sh16(vv[0:1], vv[1:2], vv[2:3])
    h = _sinpi(z0_ref[...] * (20.0 / np.pi))
    for wr, br in zip((w1, w2, w3), (b1, b2, b3)):
      h = _sinpi(_dot(wr[...], h) + br[...])
    h = jnp.concatenate([h, v_sh], axis=0)
    for wr, br in zip((r0, r1, r2), (s0, s1, s2)):
      h = _sinpi(_dot(wr[...], h) + br[...])
    z = _dot(r3[...], h) + s3[...]
    out_ref[...] = 1.0 / (1.0 + jnp.exp(-z))

  grid = (n_pix // _PB,)
  full = lambda a: pl.BlockSpec(a.shape, lambda i: (0,) * a.ndim)
  wargs = []
  wspecs = []
  inv_pi = 1.0 / np.pi
  scales = [inv_pi] * 6 + [1.0]
  for (W, b), sc in zip(list(zip(lm_Ws[1:], lm_bs[1:]))
                        + list(zip(rf_Ws, rf_bs)), scales):
    b2 = (b * sc).reshape(-1, 1).astype(jnp.float32)
    wargs += [(W * sc).astype(jnp.float32), b2]
    wspecs += [full(W), full(b2)]

  return pl.pallas_call(
      body,
      grid=grid,
      in_specs=[
          pl.BlockSpec((64, _PB), lambda i: (0, i)),
          pl.BlockSpec((3, _PB), lambda i: (0, i)),
          *wspecs,
      ],
      out_specs=pl.BlockSpec((3, _PB), lambda i: (0, i)),
      out_shape=jax.ShapeDtypeStruct((3, n_pix), jnp.float32),
  )(z0T, v3, *wargs)


def _pack_table(emb):
  b16 = lax.bitcast_convert_type(emb.astype(jnp.bfloat16), jnp.uint16)
  word = b16[:, 0].astype(jnp.uint32) | (b16[:, 1].astype(jnp.uint32) << 16)
  return lax.bitcast_convert_type(word, jnp.int32)


def kernel(uv, n, v, emb0, emb1, emb2, emb3,
           lm_W0, lm_b0, lm_W1, lm_b1, lm_W2, lm_b2, lm_W3, lm_b3,
           rf_W0, rf_b0, rf_W1, rf_b1, rf_W2, rf_b2, rf_W3, rf_b3):
  B, W, H = uv.shape[:3]
  n_pix = B * W * H

  u_flat = uv[..., 0].reshape(n_pix)
  v_flat = uv[..., 1].reshape(n_pix)
  tabs = jnp.stack([_pack_table(e) for e in (emb0, emb1, emb2, emb3)])

  nT = n.reshape(n_pix, 3).T
  vT = v.reshape(n_pix, 3).T

  chunks = 4
  m = n_pix // chunks
  outs = []
  for c in range(chunks):
    sl = slice(c * m, (c + 1) * m)
    feats = _sc_hashgrid(u_flat[sl], v_flat[sl], tabs, m)
    n_shT = _sh16(nT[0:1, sl], nT[1:2, sl], nT[2:3, sl])
    x0T = jnp.concatenate([n_shT, feats], axis=0)
    z0T = lm_W0 @ x0T + lm_b0[:, None]
    outs.append(_tc_mlp(
        z0T, vT[:, sl],
        (lm_W0, lm_W1, lm_W2, lm_W3), (lm_b0, lm_b1, lm_b2, lm_b3),
        (rf_W0, rf_W1, rf_W2, rf_W3), (rf_b0, rf_b1, rf_b2, rf_b3), m))
  out = jnp.concatenate(outs, axis=1)
  return out.T.reshape(B, W, H, 3)
